# baseline (speedup 1.0000x reference)
import functools

import jax
import jax.numpy as jnp
from jax import lax
from jax.experimental import pallas as pl
from jax.experimental.pallas import tpu as pltpu
from jax.experimental.pallas import tpu_sc as plsc

B = 16384
L = 30
TABLE = 640
HIDDEN = 256
W2 = 2 * TABLE
WP = W2 // 2

NC, NS, LANES = 2, 16, 16
NW = NC * NS
ROWS_PER_W = B // NW
CHUNK = 32
NCHUNK = ROWS_PER_W // CHUNK

B_BLK = 512
NBLK = B // B_BLK


def _sc_hist_body(idx_hbm, out_hbm, idx_v, cnt_a, cnt_b, sem_a, sem_b):
    wid = lax.axis_index("s") * NC + lax.axis_index("c")
    base = wid * ROWS_PER_W
    pltpu.sync_copy(idx_hbm.at[:, pl.ds(base * 1, ROWS_PER_W)], idx_v)

    iota16 = lax.broadcasted_iota(jnp.int32, (LANES,), 0)
    one = jnp.full((LANES,), 1, jnp.int32)
    hi_one = jnp.full((LANES,), 1 << 16, jnp.int32)
    zeros16 = jnp.zeros((LANES,), jnp.int32)

    def zero_buf(buf):
        n_per_row = WP // LANES
        unroll = 8

        def body(r, _):
            def inner(k, _):
                for u in range(unroll):
                    buf[r, pl.ds(k * (LANES * unroll) + u * LANES, LANES)] = zeros16
                return 0
            lax.fori_loop(0, n_per_row // unroll, inner, 0)
            return 0

        lax.fori_loop(0, CHUNK, body, 0)

    def scatter_chunk(buf, c):
        def body(l, _):
            off = jnp.where(l >= L, TABLE, 0).astype(jnp.int32)
            for j in range(CHUNK // LANES):
                rowvec = iota16 + j * LANES
                vec = idx_v[l, pl.ds(c * CHUNK + j * LANES, LANES)] + off
                word = lax.shift_right_logical(vec, 1)
                val = jnp.where(jnp.bitwise_and(vec, 1) == 0, one, hi_one)
                plsc.addupdate_scatter(buf, [rowvec, word], val)
            return 0

        lax.fori_loop(0, 2 * L, body, 0)

    pending = [None, None]
    for c in range(NCHUNK):
        buf, sem = (cnt_a, sem_a) if c % 2 == 0 else (cnt_b, sem_b)
        if pending[c % 2] is not None:
            pending[c % 2].wait()
        zero_buf(buf)
        scatter_chunk(buf, c)
        cp = pltpu.make_async_copy(
            buf, out_hbm.at[pl.ds(base + c * CHUNK, CHUNK), :], sem)
        cp.start()
        pending[c % 2] = cp
    pending[0].wait()
    pending[1].wait()


_sc_hist = functools.partial(
    pl.kernel,
    out_type=jax.ShapeDtypeStruct((B, WP), jnp.int32),
    mesh=plsc.VectorSubcoreMesh(core_axis_name="c", subcore_axis_name="s"),
    compiler_params=pltpu.CompilerParams(needs_layout_passes=False),
    scratch_types=[
        pltpu.VMEM((2 * L, ROWS_PER_W), jnp.int32),
        pltpu.VMEM((CHUNK, WP), jnp.int32),
        pltpu.VMEM((CHUNK, WP), jnp.int32),
        pltpu.SemaphoreType.DMA,
        pltpu.SemaphoreType.DMA,
    ],
)(_sc_hist_body)


def _mlp_block(cnt_ref, weven_ref, wodd_ref, fc2_wt_ref, fc2_b_ref,
               fc3_wt_ref, fc3_b_ref, fc4_wt_ref, fc4_b_ref, out_ref):
    cnt = cnt_ref[...]
    lo = jnp.bitwise_and(cnt, 0xFFFF).astype(jnp.float32)
    hi = lax.shift_right_logical(cnt, 16).astype(jnp.float32)
    HT = TABLE // 2
    hp = jax.lax.Precision.HIGHEST
    sum0 = (jnp.dot(lo[:, :HT], weven_ref[0], precision=hp, preferred_element_type=jnp.float32)
            + jnp.dot(hi[:, :HT], wodd_ref[0], precision=hp, preferred_element_type=jnp.float32))
    sum1 = (jnp.dot(lo[:, HT:], weven_ref[1], precision=hp, preferred_element_type=jnp.float32)
            + jnp.dot(hi[:, HT:], wodd_ref[1], precision=hp, preferred_element_type=jnp.float32))
    h = jnp.concatenate([jnp.maximum(sum0, 0.0), jnp.maximum(sum1, 0.0)], axis=1)
    h = jnp.maximum(jnp.dot(h, fc2_wt_ref[...], preferred_element_type=jnp.float32)
                    + fc2_b_ref[...], 0.0)
    h = jnp.maximum(jnp.dot(h, fc3_wt_ref[...], preferred_element_type=jnp.float32)
                    + fc3_b_ref[...], 0.0)
    out = jnp.dot(h, fc4_wt_ref[...], preferred_element_type=jnp.float32) + fc4_b_ref[...]
    out_ref[...] = out


@jax.jit
def kernel(idx0_batch, idx1_batch, w1, fc2_w, fc2_b, fc3_w, fc3_b, fc4_w, fc4_b):
    idx_t = jnp.concatenate([idx0_batch, idx1_batch], axis=1).T
    counts = _sc_hist(idx_t)
    w_even = w1[:, 0::2, :]
    w_odd = w1[:, 1::2, :]
    out = pl.pallas_call(
        _mlp_block,
        grid=(NBLK,),
        in_specs=[
            pl.BlockSpec((B_BLK, WP), lambda i: (i, 0)),
            pl.BlockSpec((2, TABLE // 2, HIDDEN), lambda i: (0, 0, 0)),
            pl.BlockSpec((2, TABLE // 2, HIDDEN), lambda i: (0, 0, 0)),
            pl.BlockSpec((2 * HIDDEN, 32), lambda i: (0, 0)),
            pl.BlockSpec((1, 32), lambda i: (0, 0)),
            pl.BlockSpec((32, 32), lambda i: (0, 0)),
            pl.BlockSpec((1, 32), lambda i: (0, 0)),
            pl.BlockSpec((32, 1), lambda i: (0, 0)),
            pl.BlockSpec((1, 1), lambda i: (0, 0)),
        ],
        out_specs=pl.BlockSpec((B_BLK, 1), lambda i: (i, 0)),
        out_shape=jax.ShapeDtypeStruct((B, 1), jnp.float32),
    )(counts, w_even, w_odd,
      fc2_w.T, fc2_b.reshape(1, 32), fc3_w.T, fc3_b.reshape(1, 32),
      fc4_w.T, fc4_b.reshape(1, 1))
    return out[:, 0]


# speedup vs baseline: 1.7251x; 1.7251x over previous
import functools

import jax
import jax.numpy as jnp
from jax import lax
from jax.experimental import pallas as pl
from jax.experimental.pallas import tpu as pltpu
from jax.experimental.pallas import tpu_sc as plsc

B = 16384
L = 30
TABLE = 640
HIDDEN = 256
W2 = 2 * TABLE
WP = W2 // 2

NC, NS, LANES = 2, 16, 16
NW = NC * NS
ROWS_PER_W = B // NW
CHUNK = 32
NCHUNK = ROWS_PER_W // CHUNK

B_BLK = 512
NBLK = B // B_BLK


def _sc_hist_body(idx_hbm, out_hbm, idx_v, cnt_a, cnt_b, sem_a, sem_b):
    wid = lax.axis_index("s") * NC + lax.axis_index("c")
    base = wid * ROWS_PER_W
    pltpu.sync_copy(idx_hbm.at[:, pl.ds(base * 1, ROWS_PER_W)], idx_v)

    iota16 = lax.broadcasted_iota(jnp.int32, (LANES,), 0)
    one = jnp.full((LANES,), 1, jnp.int32)
    hi_one = jnp.full((LANES,), 1 << 16, jnp.int32)
    zeros16 = jnp.zeros((LANES,), jnp.int32)

    def zero_buf(buf):
        n_per_row = WP // LANES
        unroll = 8

        def body(r, _):
            def inner(k, _):
                for u in range(unroll):
                    buf[r, pl.ds(k * (LANES * unroll) + u * LANES, LANES)] = zeros16
                return 0
            lax.fori_loop(0, n_per_row // unroll, inner, 0)
            return 0

        lax.fori_loop(0, CHUNK, body, 0)

    def scatter_chunk(buf, c):
        def body(l, _):
            off = jnp.where(l >= L, TABLE, 0).astype(jnp.int32)
            for j in range(CHUNK // LANES):
                rowvec = iota16 + j * LANES
                vec = idx_v[l, pl.ds(c * CHUNK + j * LANES, LANES)] + off
                word = lax.shift_right_logical(vec, 1)
                val = jnp.where(jnp.bitwise_and(vec, 1) == 0, one, hi_one)
                plsc.addupdate_scatter(buf, [rowvec, word], val)
            return 0

        lax.fori_loop(0, 2 * L, body, 0)

    pending = [None, None]
    for c in range(NCHUNK):
        buf, sem = (cnt_a, sem_a) if c % 2 == 0 else (cnt_b, sem_b)
        if pending[c % 2] is not None:
            pending[c % 2].wait()
        zero_buf(buf)
        scatter_chunk(buf, c)
        cp = pltpu.make_async_copy(
            buf, out_hbm.at[pl.ds(base + c * CHUNK, CHUNK), :], sem)
        cp.start()
        pending[c % 2] = cp
    pending[0].wait()
    pending[1].wait()


_sc_hist = functools.partial(
    pl.kernel,
    out_type=jax.ShapeDtypeStruct((B, WP), jnp.int32),
    mesh=plsc.VectorSubcoreMesh(core_axis_name="c", subcore_axis_name="s"),
    compiler_params=pltpu.CompilerParams(needs_layout_passes=False),
    scratch_types=[
        pltpu.VMEM((2 * L, ROWS_PER_W), jnp.int32),
        pltpu.VMEM((CHUNK, WP), jnp.int32),
        pltpu.VMEM((CHUNK, WP), jnp.int32),
        pltpu.SemaphoreType.DMA,
        pltpu.SemaphoreType.DMA,
    ],
)(_sc_hist_body)


def _mlp_block(cnt_ref, wph_ref, wpl_ref, fc2_wt_ref, fc2_b_ref,
               fc3_wt_ref, fc3_b_ref, fc4_wt_ref, fc4_b_ref, out_ref):
    cnt = cnt_ref[...]
    lo = jnp.bitwise_and(cnt, 0xFFFF).astype(jnp.bfloat16)
    hi = lax.shift_right_logical(cnt, 16).astype(jnp.bfloat16)
    HT = TABLE // 2
    lhs0 = jnp.concatenate([lo[:, :HT], hi[:, :HT]], axis=1)
    lhs1 = jnp.concatenate([lo[:, HT:], hi[:, HT:]], axis=1)
    sum0 = (jnp.dot(lhs0, wph_ref[0], preferred_element_type=jnp.float32)
            + jnp.dot(lhs0, wpl_ref[0], preferred_element_type=jnp.float32))
    sum1 = (jnp.dot(lhs1, wph_ref[1], preferred_element_type=jnp.float32)
            + jnp.dot(lhs1, wpl_ref[1], preferred_element_type=jnp.float32))
    h = jnp.concatenate([jnp.maximum(sum0, 0.0), jnp.maximum(sum1, 0.0)], axis=1)
    h = jnp.maximum(jnp.dot(h, fc2_wt_ref[...], preferred_element_type=jnp.float32)
                    + fc2_b_ref[...], 0.0)
    h = jnp.maximum(jnp.dot(h, fc3_wt_ref[...], preferred_element_type=jnp.float32)
                    + fc3_b_ref[...], 0.0)
    out = jnp.dot(h, fc4_wt_ref[...], preferred_element_type=jnp.float32) + fc4_b_ref[...]
    out_ref[...] = out


@jax.jit
def kernel(idx0_batch, idx1_batch, w1, fc2_w, fc2_b, fc3_w, fc3_b, fc4_w, fc4_b):
    idx_t = jnp.concatenate([idx0_batch, idx1_batch], axis=1).T
    counts = _sc_hist(idx_t)
    w_perm = jnp.concatenate([w1[:, 0::2, :], w1[:, 1::2, :]], axis=1)
    w_perm_hi = w_perm.astype(jnp.bfloat16)
    w_perm_lo = (w_perm - w_perm_hi.astype(jnp.float32)).astype(jnp.bfloat16)
    out = pl.pallas_call(
        _mlp_block,
        grid=(NBLK,),
        in_specs=[
            pl.BlockSpec((B_BLK, WP), lambda i: (i, 0)),
            pl.BlockSpec((2, TABLE, HIDDEN), lambda i: (0, 0, 0)),
            pl.BlockSpec((2, TABLE, HIDDEN), lambda i: (0, 0, 0)),
            pl.BlockSpec((2 * HIDDEN, 32), lambda i: (0, 0)),
            pl.BlockSpec((1, 32), lambda i: (0, 0)),
            pl.BlockSpec((32, 32), lambda i: (0, 0)),
            pl.BlockSpec((1, 32), lambda i: (0, 0)),
            pl.BlockSpec((32, 1), lambda i: (0, 0)),
            pl.BlockSpec((1, 1), lambda i: (0, 0)),
        ],
        out_specs=pl.BlockSpec((B_BLK, 1), lambda i: (i, 0)),
        out_shape=jax.ShapeDtypeStruct((B, 1), jnp.float32),
    )(counts, w_perm_hi, w_perm_lo,
      fc2_w.T, fc2_b.reshape(1, 32), fc3_w.T, fc3_b.reshape(1, 32),
      fc4_w.T, fc4_b.reshape(1, 1))
    return out[:, 0]


# speedup vs baseline: 1.9095x; 1.1069x over previous
import functools

import jax
import jax.numpy as jnp
from jax import lax
from jax.experimental import pallas as pl
from jax.experimental.pallas import tpu as pltpu
from jax.experimental.pallas import tpu_sc as plsc

B = 16384
NHALF = 2
BH = B // NHALF
L = 30
TABLE = 640
HIDDEN = 256
W2 = 2 * TABLE
WP = W2 // 2
HT = TABLE // 2

NC, NS, LANES = 2, 16, 16
NW = NC * NS
ROWS_PER_W = BH // NW
CHUNK = 32
NCHUNK = ROWS_PER_W // CHUNK

B_BLK = 512
NBLK = BH // B_BLK


def _sc_hist_body(idx_hbm, out_hbm, idx_v, cnt_a, cnt_b, sem_a, sem_b):
    wid = lax.axis_index("s") * NC + lax.axis_index("c")
    base = wid * ROWS_PER_W
    pltpu.sync_copy(idx_hbm.at[:, pl.ds(base * 1, ROWS_PER_W)], idx_v)

    iota16 = lax.broadcasted_iota(jnp.int32, (LANES,), 0)
    one = jnp.full((LANES,), 1, jnp.int32)
    hi_one = jnp.full((LANES,), 1 << 16, jnp.int32)
    zeros16 = jnp.zeros((LANES,), jnp.int32)

    def zero_buf(buf):
        n_per_row = WP // LANES
        unroll = 8

        def body(r, _):
            def inner(k, _):
                for u in range(unroll):
                    buf[r, pl.ds(k * (LANES * unroll) + u * LANES, LANES)] = zeros16
                return 0
            lax.fori_loop(0, n_per_row // unroll, inner, 0)
            return 0

        lax.fori_loop(0, CHUNK, body, 0)

    def scatter_chunk(buf, c):
        def body(l, _):
            toff = jnp.where(l >= L, HT, 0).astype(jnp.int32)
            for j in range(CHUNK // LANES):
                rowvec = iota16 + j * LANES
                vec = idx_v[l, pl.ds(c * CHUNK + j * LANES, LANES)]
                in_hi = vec >= HT
                word = jnp.where(in_hi, vec - HT, vec) + toff
                val = jnp.where(in_hi, hi_one, one)
                plsc.addupdate_scatter(buf, [rowvec, word], val)
            return 0

        lax.fori_loop(0, 2 * L, body, 0)

    pending = [None, None]
    for c in range(NCHUNK):
        buf, sem = (cnt_a, sem_a) if c % 2 == 0 else (cnt_b, sem_b)
        if pending[c % 2] is not None:
            pending[c % 2].wait()
        zero_buf(buf)
        scatter_chunk(buf, c)
        cp = pltpu.make_async_copy(
            buf, out_hbm.at[pl.ds(base + c * CHUNK, CHUNK), :], sem)
        cp.start()
        pending[c % 2] = cp
    pending[0].wait()
    pending[1].wait()


_sc_hist = functools.partial(
    pl.kernel,
    out_type=jax.ShapeDtypeStruct((BH, WP), jnp.int32),
    mesh=plsc.VectorSubcoreMesh(core_axis_name="c", subcore_axis_name="s"),
    compiler_params=pltpu.CompilerParams(needs_layout_passes=False),
    scratch_types=[
        pltpu.VMEM((2 * L, ROWS_PER_W), jnp.int32),
        pltpu.VMEM((CHUNK, WP), jnp.int32),
        pltpu.VMEM((CHUNK, WP), jnp.int32),
        pltpu.SemaphoreType.DMA,
        pltpu.SemaphoreType.DMA,
    ],
)(_sc_hist_body)


def _mlp_block(cnt_ref, wh_ref, wl_ref, fc2_wt_ref, fc2_b_ref,
               fc3_wt_ref, fc3_b_ref, fc4_wt_ref, fc4_b_ref, out_ref):
    cnt = cnt_ref[...]
    lo = jnp.bitwise_and(cnt, 0xFFFF).astype(jnp.bfloat16)
    hi = lax.shift_right_logical(cnt, 16).astype(jnp.bfloat16)
    lhs0 = jnp.concatenate([lo[:, :HT], hi[:, :HT]], axis=1)
    lhs1 = jnp.concatenate([lo[:, HT:], hi[:, HT:]], axis=1)
    sum0 = (jnp.dot(lhs0, wh_ref[0], preferred_element_type=jnp.float32)
            + jnp.dot(lhs0, wl_ref[0], preferred_element_type=jnp.float32))
    sum1 = (jnp.dot(lhs1, wh_ref[1], preferred_element_type=jnp.float32)
            + jnp.dot(lhs1, wl_ref[1], preferred_element_type=jnp.float32))
    h = jnp.concatenate([jnp.maximum(sum0, 0.0), jnp.maximum(sum1, 0.0)], axis=1)
    h = jnp.maximum(jnp.dot(h, fc2_wt_ref[...], preferred_element_type=jnp.float32)
                    + fc2_b_ref[...], 0.0)
    h = jnp.maximum(jnp.dot(h, fc3_wt_ref[...], preferred_element_type=jnp.float32)
                    + fc3_b_ref[...], 0.0)
    out = jnp.dot(h, fc4_wt_ref[...], preferred_element_type=jnp.float32) + fc4_b_ref[...]
    out_ref[...] = jnp.reshape(out, (1, 1, B_BLK))


def _mlp_half(counts, wh, wl, fc2_wt, fc2_b, fc3_wt, fc3_b, fc4_wt, fc4_b):
    out = pl.pallas_call(
        _mlp_block,
        grid=(NBLK,),
        in_specs=[
            pl.BlockSpec((B_BLK, WP), lambda i: (i, 0)),
            pl.BlockSpec((2, TABLE, HIDDEN), lambda i: (0, 0, 0)),
            pl.BlockSpec((2, TABLE, HIDDEN), lambda i: (0, 0, 0)),
            pl.BlockSpec((2 * HIDDEN, 32), lambda i: (0, 0)),
            pl.BlockSpec((1, 32), lambda i: (0, 0)),
            pl.BlockSpec((32, 32), lambda i: (0, 0)),
            pl.BlockSpec((1, 32), lambda i: (0, 0)),
            pl.BlockSpec((32, 1), lambda i: (0, 0)),
            pl.BlockSpec((1, 1), lambda i: (0, 0)),
        ],
        out_specs=pl.BlockSpec((1, 1, B_BLK), lambda i: (i, 0, 0)),
        out_shape=jax.ShapeDtypeStruct((NBLK, 1, B_BLK), jnp.float32),
    )(counts, wh, wl, fc2_wt, fc2_b, fc3_wt, fc3_b, fc4_wt, fc4_b)
    return out.reshape(BH)


@jax.jit
def kernel(idx0_batch, idx1_batch, w1, fc2_w, fc2_b, fc3_w, fc3_b, fc4_w, fc4_b):
    idx_t = jnp.concatenate([idx0_batch, idx1_batch], axis=1).T
    w_hi = w1.astype(jnp.bfloat16)
    w_lo = (w1 - w_hi.astype(jnp.float32)).astype(jnp.bfloat16)
    fc2_wt = fc2_w.T
    fc2_bb = fc2_b.reshape(1, 32)
    fc3_wt = fc3_w.T
    fc3_bb = fc3_b.reshape(1, 32)
    fc4_wt = fc4_w.T
    fc4_bb = fc4_b.reshape(1, 1)
    outs = []
    for hf in range(NHALF):
        counts = _sc_hist(idx_t[:, hf * BH:(hf + 1) * BH])
        outs.append(_mlp_half(counts, w_hi, w_lo, fc2_wt, fc2_bb,
                              fc3_wt, fc3_bb, fc4_wt, fc4_bb))
    return jnp.concatenate(outs)


# speedup vs baseline: 1.9536x; 1.0231x over previous
import functools

import jax
import jax.numpy as jnp
from jax import lax
from jax.experimental import pallas as pl
from jax.experimental.pallas import tpu as pltpu
from jax.experimental.pallas import tpu_sc as plsc

B = 16384
NHALF = 2
BH = B // NHALF
L = 30
TABLE = 640
HIDDEN = 256
W2 = 2 * TABLE
WP = TABLE
HT = TABLE // 2

NC, NS, LANES = 2, 16, 16
NW = NC * NS
ROWS_PER_W = BH // NW
CHUNK = 32
NCHUNK = ROWS_PER_W // CHUNK

B_BLK = 512
NBLK = BH // B_BLK


def _sc_hist_body(idx_hbm, out_hbm, idx_v, cnt_a, cnt_b, sem_a, sem_b):
    wid = lax.axis_index("s") * NC + lax.axis_index("c")
    base = wid * ROWS_PER_W
    pltpu.sync_copy(idx_hbm.at[:, pl.ds(base * 1, ROWS_PER_W)], idx_v)

    iota16 = lax.broadcasted_iota(jnp.int32, (LANES,), 0)
    one = jnp.full((LANES,), 1, jnp.int32)
    hi_one = jnp.full((LANES,), 1 << 16, jnp.int32)
    zeros16 = jnp.zeros((LANES,), jnp.int32)

    def zero_buf(buf):
        n_per_row = WP // LANES
        unroll = 8

        def body(r, _):
            def inner(k, _):
                for u in range(unroll):
                    buf[r, pl.ds(k * (LANES * unroll) + u * LANES, LANES)] = zeros16
                return 0
            lax.fori_loop(0, n_per_row // unroll, inner, 0)
            return 0

        lax.fori_loop(0, CHUNK, body, 0)

    def scatter_chunk(buf, c):
        def body(l, _):
            val = jnp.where(l < L, one, hi_one)
            for j in range(CHUNK // LANES):
                rowvec = iota16 + j * LANES
                vec = idx_v[l, pl.ds(c * CHUNK + j * LANES, LANES)]
                plsc.addupdate_scatter(buf, [rowvec, vec], val)
            return 0

        lax.fori_loop(0, 2 * L, body, 0)

    pending = [None, None]
    for c in range(NCHUNK):
        buf, sem = (cnt_a, sem_a) if c % 2 == 0 else (cnt_b, sem_b)
        if pending[c % 2] is not None:
            pending[c % 2].wait()
        zero_buf(buf)
        scatter_chunk(buf, c)
        cp = pltpu.make_async_copy(
            buf, out_hbm.at[pl.ds(base + c * CHUNK, CHUNK), :], sem)
        cp.start()
        pending[c % 2] = cp
    pending[0].wait()
    pending[1].wait()


_sc_hist = functools.partial(
    pl.kernel,
    out_type=jax.ShapeDtypeStruct((BH, WP), jnp.int32),
    mesh=plsc.VectorSubcoreMesh(core_axis_name="c", subcore_axis_name="s"),
    compiler_params=pltpu.CompilerParams(needs_layout_passes=False),
    scratch_types=[
        pltpu.VMEM((2 * L, ROWS_PER_W), jnp.int32),
        pltpu.VMEM((CHUNK, WP), jnp.int32),
        pltpu.VMEM((CHUNK, WP), jnp.int32),
        pltpu.SemaphoreType.DMA,
        pltpu.SemaphoreType.DMA,
    ],
)(_sc_hist_body)


def _mlp_block(cnt_ref, wh_ref, wl_ref, fc2_wt_ref, fc2_b_ref,
               fc3_wt_ref, fc3_b_ref, fc4_wt_ref, fc4_b_ref, out_ref):
    cnt = cnt_ref[...]
    lhs0 = jnp.bitwise_and(cnt, 0xFFFF).astype(jnp.bfloat16)
    lhs1 = lax.shift_right_logical(cnt, 16).astype(jnp.bfloat16)
    sum0 = (jnp.dot(lhs0, wh_ref[0], preferred_element_type=jnp.float32)
            + jnp.dot(lhs0, wl_ref[0], preferred_element_type=jnp.float32))
    sum1 = (jnp.dot(lhs1, wh_ref[1], preferred_element_type=jnp.float32)
            + jnp.dot(lhs1, wl_ref[1], preferred_element_type=jnp.float32))
    h = jnp.concatenate([jnp.maximum(sum0, 0.0), jnp.maximum(sum1, 0.0)], axis=1)
    h = jnp.maximum(jnp.dot(h, fc2_wt_ref[...], preferred_element_type=jnp.float32)
                    + fc2_b_ref[...], 0.0)
    h = jnp.maximum(jnp.dot(h, fc3_wt_ref[...], preferred_element_type=jnp.float32)
                    + fc3_b_ref[...], 0.0)
    out = jnp.dot(h, fc4_wt_ref[...], preferred_element_type=jnp.float32) + fc4_b_ref[...]
    out_ref[...] = jnp.reshape(out, (1, 1, B_BLK))


def _mlp_half(counts, wh, wl, fc2_wt, fc2_b, fc3_wt, fc3_b, fc4_wt, fc4_b):
    out = pl.pallas_call(
        _mlp_block,
        grid=(NBLK,),
        in_specs=[
            pl.BlockSpec((B_BLK, WP), lambda i: (i, 0)),
            pl.BlockSpec((2, TABLE, HIDDEN), lambda i: (0, 0, 0)),
            pl.BlockSpec((2, TABLE, HIDDEN), lambda i: (0, 0, 0)),
            pl.BlockSpec((2 * HIDDEN, 32), lambda i: (0, 0)),
            pl.BlockSpec((1, 32), lambda i: (0, 0)),
            pl.BlockSpec((32, 32), lambda i: (0, 0)),
            pl.BlockSpec((1, 32), lambda i: (0, 0)),
            pl.BlockSpec((32, 1), lambda i: (0, 0)),
            pl.BlockSpec((1, 1), lambda i: (0, 0)),
        ],
        out_specs=pl.BlockSpec((1, 1, B_BLK), lambda i: (i, 0, 0)),
        out_shape=jax.ShapeDtypeStruct((NBLK, 1, B_BLK), jnp.float32),
    )(counts, wh, wl, fc2_wt, fc2_b, fc3_wt, fc3_b, fc4_wt, fc4_b)
    return out.reshape(BH)


@jax.jit
def kernel(idx0_batch, idx1_batch, w1, fc2_w, fc2_b, fc3_w, fc3_b, fc4_w, fc4_b):
    idx_t = jnp.concatenate([idx0_batch, idx1_batch], axis=1).T
    w_hi = w1.astype(jnp.bfloat16)
    w_lo = (w1 - w_hi.astype(jnp.float32)).astype(jnp.bfloat16)
    fc2_wt = fc2_w.T
    fc2_bb = fc2_b.reshape(1, 32)
    fc3_wt = fc3_w.T
    fc3_bb = fc3_b.reshape(1, 32)
    fc4_wt = fc4_w.T
    fc4_bb = fc4_b.reshape(1, 1)
    outs = []
    for hf in range(NHALF):
        counts = _sc_hist(idx_t[:, hf * BH:(hf + 1) * BH])
        outs.append(_mlp_half(counts, w_hi, w_lo, fc2_wt, fc2_bb,
                              fc3_wt, fc3_bb, fc4_wt, fc4_bb))
    return jnp.concatenate(outs)


# speedup vs baseline: 2.1115x; 1.0809x over previous
import functools

import jax
import jax.numpy as jnp
from jax import lax
from jax.experimental import pallas as pl
from jax.experimental.pallas import tpu as pltpu
from jax.experimental.pallas import tpu_sc as plsc

B = 16384
NHALF = 2
BH = B // NHALF
L = 30
TABLE = 640
HIDDEN = 256
W2 = 2 * TABLE
WP = TABLE
HT = TABLE // 2

NC, NS, LANES = 2, 16, 16
NW = NC * NS
ROWS_PER_W = BH // NW
CHUNK = 32
NCHUNK = ROWS_PER_W // CHUNK

B_BLK = 1024
NBLK = BH // B_BLK


def _sc_hist_body(idx_hbm, out_hbm, idx_v, cnt_a, cnt_b, sem_a, sem_b):
    wid = lax.axis_index("s") * NC + lax.axis_index("c")
    base = wid * ROWS_PER_W
    pltpu.sync_copy(idx_hbm.at[:, pl.ds(base * 1, ROWS_PER_W)], idx_v)

    iota16 = lax.broadcasted_iota(jnp.int32, (LANES,), 0)
    one = jnp.full((LANES,), 1, jnp.int32)
    hi_one = jnp.full((LANES,), 1 << 16, jnp.int32)
    zeros16 = jnp.zeros((LANES,), jnp.int32)

    def zero_buf(buf):
        n_per_row = WP // LANES
        unroll = 8

        def body(r, _):
            def inner(k, _):
                for u in range(unroll):
                    buf[r, pl.ds(k * (LANES * unroll) + u * LANES, LANES)] = zeros16
                return 0
            lax.fori_loop(0, n_per_row // unroll, inner, 0)
            return 0

        lax.fori_loop(0, CHUNK, body, 0)

    def scatter_chunk(buf, c):
        def body(l, _):
            val = jnp.where(l < L, one, hi_one)
            for j in range(CHUNK // LANES):
                rowvec = iota16 + j * LANES
                vec = idx_v[l, pl.ds(c * CHUNK + j * LANES, LANES)]
                plsc.addupdate_scatter(buf, [rowvec, vec], val)
            return 0

        lax.fori_loop(0, 2 * L, body, 0)

    pending = [None, None]
    for c in range(NCHUNK):
        buf, sem = (cnt_a, sem_a) if c % 2 == 0 else (cnt_b, sem_b)
        if pending[c % 2] is not None:
            pending[c % 2].wait()
        zero_buf(buf)
        scatter_chunk(buf, c)
        cp = pltpu.make_async_copy(
            buf, out_hbm.at[pl.ds(base + c * CHUNK, CHUNK), :], sem)
        cp.start()
        pending[c % 2] = cp
    pending[0].wait()
    pending[1].wait()


_sc_hist = functools.partial(
    pl.kernel,
    out_type=jax.ShapeDtypeStruct((BH, WP), jnp.int32),
    mesh=plsc.VectorSubcoreMesh(core_axis_name="c", subcore_axis_name="s"),
    compiler_params=pltpu.CompilerParams(needs_layout_passes=False),
    scratch_types=[
        pltpu.VMEM((2 * L, ROWS_PER_W), jnp.int32),
        pltpu.VMEM((CHUNK, WP), jnp.int32),
        pltpu.VMEM((CHUNK, WP), jnp.int32),
        pltpu.SemaphoreType.DMA,
        pltpu.SemaphoreType.DMA,
    ],
)(_sc_hist_body)


def _mlp_block(cnt_ref, wh_ref, wl_ref, fc2_wt_ref, fc2_b_ref,
               fc3_wt_ref, fc3_b_ref, fc4_wt_ref, fc4_b_ref, out_ref):
    cnt = cnt_ref[...]
    lhs0 = jnp.bitwise_and(cnt, 0xFFFF).astype(jnp.bfloat16)
    lhs1 = lax.shift_right_logical(cnt, 16).astype(jnp.bfloat16)
    sum0 = (jnp.dot(lhs0, wh_ref[0], preferred_element_type=jnp.float32)
            + jnp.dot(lhs0, wl_ref[0], preferred_element_type=jnp.float32))
    sum1 = (jnp.dot(lhs1, wh_ref[1], preferred_element_type=jnp.float32)
            + jnp.dot(lhs1, wl_ref[1], preferred_element_type=jnp.float32))
    h = jnp.concatenate([jnp.maximum(sum0, 0.0), jnp.maximum(sum1, 0.0)], axis=1)
    h = jnp.maximum(jnp.dot(h, fc2_wt_ref[...], preferred_element_type=jnp.float32)
                    + fc2_b_ref[...], 0.0)
    h = jnp.maximum(jnp.dot(h, fc3_wt_ref[...], preferred_element_type=jnp.float32)
                    + fc3_b_ref[...], 0.0)
    out = jnp.dot(h, fc4_wt_ref[...], preferred_element_type=jnp.float32) + fc4_b_ref[...]
    out_ref[...] = jnp.reshape(out, (1, 1, B_BLK))


def _mlp_half(counts, wh, wl, fc2_wt, fc2_b, fc3_wt, fc3_b, fc4_wt, fc4_b):
    out = pl.pallas_call(
        _mlp_block,
        grid=(NBLK,),
        in_specs=[
            pl.BlockSpec((B_BLK, WP), lambda i: (i, 0)),
            pl.BlockSpec((2, TABLE, HIDDEN), lambda i: (0, 0, 0)),
            pl.BlockSpec((2, TABLE, HIDDEN), lambda i: (0, 0, 0)),
            pl.BlockSpec((2 * HIDDEN, 32), lambda i: (0, 0)),
            pl.BlockSpec((1, 32), lambda i: (0, 0)),
            pl.BlockSpec((32, 32), lambda i: (0, 0)),
            pl.BlockSpec((1, 32), lambda i: (0, 0)),
            pl.BlockSpec((32, 1), lambda i: (0, 0)),
            pl.BlockSpec((1, 1), lambda i: (0, 0)),
        ],
        out_specs=pl.BlockSpec((1, 1, B_BLK), lambda i: (i, 0, 0)),
        out_shape=jax.ShapeDtypeStruct((NBLK, 1, B_BLK), jnp.float32),
    )(counts, wh, wl, fc2_wt, fc2_b, fc3_wt, fc3_b, fc4_wt, fc4_b)
    return out.reshape(BH)


@jax.jit
def kernel(idx0_batch, idx1_batch, w1, fc2_w, fc2_b, fc3_w, fc3_b, fc4_w, fc4_b):
    idx_t = jnp.concatenate([idx0_batch, idx1_batch], axis=1).T
    w_hi = w1.astype(jnp.bfloat16)
    w_lo = (w1 - w_hi.astype(jnp.float32)).astype(jnp.bfloat16)
    fc2_wt = fc2_w.T
    fc2_bb = fc2_b.reshape(1, 32)
    fc3_wt = fc3_w.T
    fc3_bb = fc3_b.reshape(1, 32)
    fc4_wt = fc4_w.T
    fc4_bb = fc4_b.reshape(1, 1)
    outs = []
    for hf in range(NHALF):
        counts = _sc_hist(idx_t[:, hf * BH:(hf + 1) * BH])
        outs.append(_mlp_half(counts, w_hi, w_lo, fc2_wt, fc2_bb,
                              fc3_wt, fc3_bb, fc4_wt, fc4_bb))
    return jnp.concatenate(outs)


# speedup vs baseline: 2.1389x; 1.0130x over previous
import functools

import jax
import jax.numpy as jnp
from jax import lax
from jax.experimental import pallas as pl
from jax.experimental.pallas import tpu as pltpu
from jax.experimental.pallas import tpu_sc as plsc

B = 16384
NHALF = 2
BH = B // NHALF
L = 30
TABLE = 640
HIDDEN = 256
W2 = 2 * TABLE
WP = TABLE
HT = TABLE // 2

NC, NS, LANES = 2, 16, 16
NW = NC * NS
ROWS_PER_W = BH // NW
CHUNK = 32
NCHUNK = ROWS_PER_W // CHUNK

B_BLK = 2048
NBLK = BH // B_BLK


def _sc_hist_body(idx_hbm, out_hbm, idx_v, cnt_a, cnt_b, sem_a, sem_b):
    wid = lax.axis_index("s") * NC + lax.axis_index("c")
    base = wid * ROWS_PER_W
    pltpu.sync_copy(idx_hbm.at[:, pl.ds(base * 1, ROWS_PER_W)], idx_v)

    iota16 = lax.broadcasted_iota(jnp.int32, (LANES,), 0)
    one = jnp.full((LANES,), 1, jnp.int32)
    hi_one = jnp.full((LANES,), 1 << 16, jnp.int32)
    zeros16 = jnp.zeros((LANES,), jnp.int32)

    def zero_buf(buf):
        n_per_row = WP // LANES
        unroll = 8

        def body(r, _):
            def inner(k, _):
                for u in range(unroll):
                    buf[r, pl.ds(k * (LANES * unroll) + u * LANES, LANES)] = zeros16
                return 0
            lax.fori_loop(0, n_per_row // unroll, inner, 0)
            return 0

        lax.fori_loop(0, CHUNK, body, 0)

    def scatter_chunk(buf, c):
        def body(l, _):
            val = jnp.where(l < L, one, hi_one)
            for j in range(CHUNK // LANES):
                rowvec = iota16 + j * LANES
                vec = idx_v[l, pl.ds(c * CHUNK + j * LANES, LANES)]
                plsc.addupdate_scatter(buf, [rowvec, vec], val)
            return 0

        lax.fori_loop(0, 2 * L, body, 0)

    pending = [None, None]
    for c in range(NCHUNK):
        buf, sem = (cnt_a, sem_a) if c % 2 == 0 else (cnt_b, sem_b)
        if pending[c % 2] is not None:
            pending[c % 2].wait()
        zero_buf(buf)
        scatter_chunk(buf, c)
        cp = pltpu.make_async_copy(
            buf, out_hbm.at[pl.ds(base + c * CHUNK, CHUNK), :], sem)
        cp.start()
        pending[c % 2] = cp
    pending[0].wait()
    pending[1].wait()


_sc_hist = functools.partial(
    pl.kernel,
    out_type=jax.ShapeDtypeStruct((BH, WP), jnp.int32),
    mesh=plsc.VectorSubcoreMesh(core_axis_name="c", subcore_axis_name="s"),
    compiler_params=pltpu.CompilerParams(needs_layout_passes=False),
    scratch_types=[
        pltpu.VMEM((2 * L, ROWS_PER_W), jnp.int32),
        pltpu.VMEM((CHUNK, WP), jnp.int32),
        pltpu.VMEM((CHUNK, WP), jnp.int32),
        pltpu.SemaphoreType.DMA,
        pltpu.SemaphoreType.DMA,
    ],
)(_sc_hist_body)


def _mlp_block(cnt_ref, wh_ref, wl_ref, fc2_wt_ref, fc2_b_ref,
               fc3_wt_ref, fc3_b_ref, fc4_wt_ref, fc4_b_ref, out_ref):
    cnt = cnt_ref[...]
    lhs0 = jnp.bitwise_and(cnt, 0xFFFF).astype(jnp.bfloat16)
    lhs1 = lax.shift_right_logical(cnt, 16).astype(jnp.bfloat16)
    sum0 = (jnp.dot(lhs0, wh_ref[0], preferred_element_type=jnp.float32)
            + jnp.dot(lhs0, wl_ref[0], preferred_element_type=jnp.float32))
    sum1 = (jnp.dot(lhs1, wh_ref[1], preferred_element_type=jnp.float32)
            + jnp.dot(lhs1, wl_ref[1], preferred_element_type=jnp.float32))
    h = jnp.concatenate([jnp.maximum(sum0, 0.0), jnp.maximum(sum1, 0.0)], axis=1)
    h = jnp.maximum(jnp.dot(h, fc2_wt_ref[...], preferred_element_type=jnp.float32)
                    + fc2_b_ref[...], 0.0)
    h = jnp.maximum(jnp.dot(h, fc3_wt_ref[...], preferred_element_type=jnp.float32)
                    + fc3_b_ref[...], 0.0)
    out = jnp.dot(h, fc4_wt_ref[...], preferred_element_type=jnp.float32) + fc4_b_ref[...]
    out_ref[...] = jnp.reshape(out, (1, 1, B_BLK))


def _mlp_half(counts, wh, wl, fc2_wt, fc2_b, fc3_wt, fc3_b, fc4_wt, fc4_b):
    out = pl.pallas_call(
        _mlp_block,
        grid=(NBLK,),
        in_specs=[
            pl.BlockSpec((B_BLK, WP), lambda i: (i, 0)),
            pl.BlockSpec((2, TABLE, HIDDEN), lambda i: (0, 0, 0)),
            pl.BlockSpec((2, TABLE, HIDDEN), lambda i: (0, 0, 0)),
            pl.BlockSpec((2 * HIDDEN, 32), lambda i: (0, 0)),
            pl.BlockSpec((1, 32), lambda i: (0, 0)),
            pl.BlockSpec((32, 32), lambda i: (0, 0)),
            pl.BlockSpec((1, 32), lambda i: (0, 0)),
            pl.BlockSpec((32, 1), lambda i: (0, 0)),
            pl.BlockSpec((1, 1), lambda i: (0, 0)),
        ],
        out_specs=pl.BlockSpec((1, 1, B_BLK), lambda i: (i, 0, 0)),
        out_shape=jax.ShapeDtypeStruct((NBLK, 1, B_BLK), jnp.float32),
    )(counts, wh, wl, fc2_wt, fc2_b, fc3_wt, fc3_b, fc4_wt, fc4_b)
    return out.reshape(BH)


@jax.jit
def kernel(idx0_batch, idx1_batch, w1, fc2_w, fc2_b, fc3_w, fc3_b, fc4_w, fc4_b):
    idx_t = jnp.concatenate([idx0_batch, idx1_batch], axis=1).T
    w_hi = w1.astype(jnp.bfloat16)
    w_lo = (w1 - w_hi.astype(jnp.float32)).astype(jnp.bfloat16)
    fc2_wt = fc2_w.T
    fc2_bb = fc2_b.reshape(1, 32)
    fc3_wt = fc3_w.T
    fc3_bb = fc3_b.reshape(1, 32)
    fc4_wt = fc4_w.T
    fc4_bb = fc4_b.reshape(1, 1)
    outs = []
    for hf in range(NHALF):
        counts = _sc_hist(idx_t[:, hf * BH:(hf + 1) * BH])
        outs.append(_mlp_half(counts, w_hi, w_lo, fc2_wt, fc2_bb,
                              fc3_wt, fc3_bb, fc4_wt, fc4_bb))
    return jnp.concatenate(outs)
